# baseline (speedup 1.0000x reference)
import functools

import jax
import jax.numpy as jnp
from jax import lax
from jax.experimental import pallas as pl
from jax.experimental.pallas import tpu as pltpu
from jax.experimental.pallas import tpu_sc as plsc

N_NODES, N_EDGES, DIM = 10000, 320000, 128

NC, NS = 2, 16
NW = NC * NS
CHUNK = 128
EPT = 10240
E_PAD = NW * EPT
NCHUNK = EPT // CHUNK
N_PAD = 10240
RPT = N_PAD // NS
BLK = 1000


def _relu_body(x_ref, o_ref):
    o_ref[...] = jnp.maximum(x_ref[...], 0.0)


def _mlp_body(x_ref, a0_ref, a1_ref, w1_ref, b1_ref, w2_ref, b2_ref,
              eps_ref, o_ref):
    h = x_ref[...] * (1.0 + eps_ref[0, 0]) + a0_ref[...] + a1_ref[...]
    h = jnp.dot(h, w1_ref[...], preferred_element_type=jnp.float32)
    h = jnp.maximum(h + b1_ref[...], 0.0)
    o = jnp.dot(h, w2_ref[...], preferred_element_type=jnp.float32)
    o_ref[...] = o + b2_ref[...]


IB = 4
RB = 2
UNROLL = 4
NROUND = NCHUNK // UNROLL


def _sc_agg_body(xr_hbm, src_hbm, dst_hbm, zeros_hbm, agg_hbm, *rest):
    sidx = rest[0:IB]
    didx = rest[IB:2 * IB]
    rows = rest[2 * IB:2 * IB + RB]
    isem = rest[2 * IB + RB:3 * IB + RB]
    gsem = rest[3 * IB + RB:3 * IB + 2 * RB]
    acc = rest[3 * IB + 2 * RB]

    c = lax.axis_index("c")
    s = lax.axis_index("s")
    wid = s * NC + c
    pltpu.sync_copy(zeros_hbm, acc.at[pl.ds(s * RPT, RPT)])
    plsc.subcore_barrier()

    base = wid * EPT
    last_off = base + (NCHUNK - 1) * CHUNK

    def idx_start(off, b):
        pltpu.make_async_copy(src_hbm.at[pl.ds(off, CHUNK)], sidx[b],
                              isem[b]).start()
        pltpu.make_async_copy(dst_hbm.at[pl.ds(off, CHUNK)], didx[b],
                              isem[b]).start()

    def idx_wait(b):
        pltpu.make_async_copy(src_hbm.at[pl.ds(base, CHUNK)], sidx[b],
                              isem[b]).wait()
        pltpu.make_async_copy(dst_hbm.at[pl.ds(base, CHUNK)], didx[b],
                              isem[b]).wait()

    def g_start(b, rb):
        pltpu.make_async_copy(xr_hbm.at[sidx[b]], rows[rb], gsem[rb]).start()

    def g_wait(b, rb):
        pltpu.make_async_copy(xr_hbm.at[sidx[b]], rows[rb], gsem[rb]).wait()

    for j in range(IB):
        idx_start(base + j * CHUNK, j)
    idx_wait(0)
    g_start(0, 0)

    def round_body(r, carry):
        k0 = r * UNROLL
        for u in range(UNROLL):
            k = k0 + u
            b = u % IB
            b1 = (u + 1) % IB
            rb = u % RB
            rb1 = (u + 1) % RB
            g_wait(b, rb)
            idx_wait(b1)
            g_start(b1, rb1)
            pltpu.sync_copy(rows[rb], acc.at[didx[b]], add=True)
            off = jnp.minimum(base + (k + IB) * CHUNK, last_off)
            idx_start(off, b)
        return carry

    lax.fori_loop(0, NROUND, round_body, 0)

    g_wait(0, 0)
    for b in range(1, IB):
        idx_wait(b)

    plsc.subcore_barrier()
    pltpu.sync_copy(acc.at[pl.ds(s * RPT, RPT)],
                    agg_hbm.at[pl.ds(c * N_PAD + s * RPT, RPT)])


def kernel(x, edge_index, W1, b1, W2, b2, eps):
    npad = E_PAD - N_EDGES
    src = jnp.concatenate([edge_index[0],
                           jnp.zeros((npad,), dtype=jnp.int32)])
    dst = jnp.concatenate([edge_index[1],
                           jnp.full((npad,), N_PAD - 1, dtype=jnp.int32)])

    xr = pl.pallas_call(
        _relu_body,
        grid=(N_NODES // BLK,),
        in_specs=[pl.BlockSpec((BLK, DIM), lambda i: (i, 0))],
        out_specs=pl.BlockSpec((BLK, DIM), lambda i: (i, 0)),
        out_shape=jax.ShapeDtypeStruct((N_NODES, DIM), jnp.float32),
    )(x)

    agg_fn = pl.kernel(
        _sc_agg_body,
        out_type=jax.ShapeDtypeStruct((NC * N_PAD, DIM), jnp.float32),
        mesh=plsc.VectorSubcoreMesh(core_axis_name="c", subcore_axis_name="s"),
        scratch_types=(
            [pltpu.VMEM((CHUNK,), jnp.int32)] * IB
            + [pltpu.VMEM((CHUNK,), jnp.int32)] * IB
            + [pltpu.VMEM((CHUNK, DIM), jnp.float32)] * RB
            + [pltpu.SemaphoreType.DMA] * IB
            + [pltpu.SemaphoreType.DMA] * RB
            + [pltpu.VMEM_SHARED((N_PAD, DIM), jnp.float32)]
        ),
    )
    aggp = agg_fn(xr, src, dst, jnp.zeros((RPT, DIM), jnp.float32))
    agg0 = aggp[:N_NODES]
    agg1 = aggp[N_PAD:N_PAD + N_NODES]

    out = pl.pallas_call(
        _mlp_body,
        grid=(N_NODES // BLK,),
        in_specs=[
            pl.BlockSpec((BLK, DIM), lambda i: (i, 0)),
            pl.BlockSpec((BLK, DIM), lambda i: (i, 0)),
            pl.BlockSpec((BLK, DIM), lambda i: (i, 0)),
            pl.BlockSpec((DIM, DIM), lambda i: (0, 0)),
            pl.BlockSpec((1, DIM), lambda i: (0, 0)),
            pl.BlockSpec((DIM, DIM), lambda i: (0, 0)),
            pl.BlockSpec((1, DIM), lambda i: (0, 0)),
            pl.BlockSpec((1, 1), lambda i: (0, 0), memory_space=pltpu.SMEM),
        ],
        out_specs=pl.BlockSpec((BLK, DIM), lambda i: (i, 0)),
        out_shape=jax.ShapeDtypeStruct((N_NODES, DIM), jnp.float32),
    )(x, agg0, agg1, W1, b1.reshape(1, DIM), W2, b2.reshape(1, DIM),
      eps.reshape(1, 1))
    return out


# speedup vs baseline: 1.0457x; 1.0457x over previous
import functools

import jax
import jax.numpy as jnp
from jax import lax
from jax.experimental import pallas as pl
from jax.experimental.pallas import tpu as pltpu
from jax.experimental.pallas import tpu_sc as plsc

N_NODES, N_EDGES, DIM = 10000, 320000, 128

NC, NS = 2, 16
NW = NC * NS
CHUNK = 128
EPT = 10240
E_PAD = NW * EPT
NCHUNK = EPT // CHUNK
N_PAD = 10240
RPT = N_PAD // NS
BLK = 1000


def _relu_body(x_ref, o_ref):
    o_ref[...] = jnp.maximum(x_ref[...], 0.0)


def _mlp_body(x_ref, a0_ref, a1_ref, w1_ref, b1_ref, w2_ref, b2_ref,
              eps_ref, o_ref):
    h = x_ref[...] * (1.0 + eps_ref[0, 0]) + a0_ref[...] + a1_ref[...]
    h = jnp.dot(h, w1_ref[...], preferred_element_type=jnp.float32)
    h = jnp.maximum(h + b1_ref[...], 0.0)
    o = jnp.dot(h, w2_ref[...], preferred_element_type=jnp.float32)
    o_ref[...] = o + b2_ref[...]


IB = 4
RB = 1
UNROLL = 4
NROUND = NCHUNK // UNROLL


def _sc_agg_body(xr_hbm, src_hbm, dst_hbm, zeros_hbm, agg_hbm, *rest):
    sidx = rest[0:IB]
    didx = rest[IB:2 * IB]
    rows = rest[2 * IB:2 * IB + RB]
    isem = rest[2 * IB + RB:3 * IB + RB]
    gsem = rest[3 * IB + RB:3 * IB + 2 * RB]
    acc = rest[3 * IB + 2 * RB]

    c = lax.axis_index("c")
    s = lax.axis_index("s")
    wid = s * NC + c
    pltpu.sync_copy(zeros_hbm, acc.at[pl.ds(s * RPT, RPT)])
    plsc.subcore_barrier()

    base = wid * EPT
    last_off = base + (NCHUNK - 1) * CHUNK

    def idx_start(off, b):
        pltpu.make_async_copy(src_hbm.at[pl.ds(off, CHUNK)], sidx[b],
                              isem[b]).start()
        pltpu.make_async_copy(dst_hbm.at[pl.ds(off, CHUNK)], didx[b],
                              isem[b]).start()

    def idx_wait(b):
        pltpu.make_async_copy(src_hbm.at[pl.ds(base, CHUNK)], sidx[b],
                              isem[b]).wait()
        pltpu.make_async_copy(dst_hbm.at[pl.ds(base, CHUNK)], didx[b],
                              isem[b]).wait()

    def g_start(b, rb):
        pltpu.make_async_copy(xr_hbm.at[sidx[b]], rows[rb], gsem[rb]).start()

    def g_wait(b, rb):
        pltpu.make_async_copy(xr_hbm.at[sidx[b]], rows[rb], gsem[rb]).wait()

    for j in range(IB):
        idx_start(base + j * CHUNK, j)
    idx_wait(0)
    g_start(0, 0)

    def round_body(r, carry):
        k0 = r * UNROLL
        for u in range(UNROLL):
            k = k0 + u
            b = u % IB
            b1 = (u + 1) % IB
            rb = u % RB
            rb1 = (u + 1) % RB
            g_wait(b, rb)
            if RB > 1:
                idx_wait(b1)
                g_start(b1, rb1)
            pltpu.sync_copy(rows[rb], acc.at[didx[b]], add=True)
            if RB == 1:
                idx_wait(b1)
                g_start(b1, rb1)
            off = jnp.minimum(base + (k + IB) * CHUNK, last_off)
            idx_start(off, b)
        return carry

    lax.fori_loop(0, NROUND, round_body, 0)

    g_wait(0, 0)
    for b in range(1, IB):
        idx_wait(b)

    plsc.subcore_barrier()
    pltpu.sync_copy(acc.at[pl.ds(s * RPT, RPT)],
                    agg_hbm.at[pl.ds(c * N_PAD + s * RPT, RPT)])


def kernel(x, edge_index, W1, b1, W2, b2, eps):
    npad = E_PAD - N_EDGES
    src = jnp.concatenate([edge_index[0],
                           jnp.zeros((npad,), dtype=jnp.int32)])
    dst = jnp.concatenate([edge_index[1],
                           jnp.full((npad,), N_PAD - 1, dtype=jnp.int32)])

    xr = pl.pallas_call(
        _relu_body,
        grid=(N_NODES // BLK,),
        in_specs=[pl.BlockSpec((BLK, DIM), lambda i: (i, 0))],
        out_specs=pl.BlockSpec((BLK, DIM), lambda i: (i, 0)),
        out_shape=jax.ShapeDtypeStruct((N_NODES, DIM), jnp.float32),
    )(x)

    agg_fn = pl.kernel(
        _sc_agg_body,
        out_type=jax.ShapeDtypeStruct((NC * N_PAD, DIM), jnp.float32),
        mesh=plsc.VectorSubcoreMesh(core_axis_name="c", subcore_axis_name="s"),
        scratch_types=(
            [pltpu.VMEM((CHUNK,), jnp.int32)] * IB
            + [pltpu.VMEM((CHUNK,), jnp.int32)] * IB
            + [pltpu.VMEM((CHUNK, DIM), jnp.float32)] * RB
            + [pltpu.SemaphoreType.DMA] * IB
            + [pltpu.SemaphoreType.DMA] * RB
            + [pltpu.VMEM_SHARED((N_PAD, DIM), jnp.float32)]
        ),
    )
    aggp = agg_fn(xr, src, dst, jnp.zeros((RPT, DIM), jnp.float32))
    agg0 = aggp[:N_NODES]
    agg1 = aggp[N_PAD:N_PAD + N_NODES]

    out = pl.pallas_call(
        _mlp_body,
        grid=(N_NODES // BLK,),
        in_specs=[
            pl.BlockSpec((BLK, DIM), lambda i: (i, 0)),
            pl.BlockSpec((BLK, DIM), lambda i: (i, 0)),
            pl.BlockSpec((BLK, DIM), lambda i: (i, 0)),
            pl.BlockSpec((DIM, DIM), lambda i: (0, 0)),
            pl.BlockSpec((1, DIM), lambda i: (0, 0)),
            pl.BlockSpec((DIM, DIM), lambda i: (0, 0)),
            pl.BlockSpec((1, DIM), lambda i: (0, 0)),
            pl.BlockSpec((1, 1), lambda i: (0, 0), memory_space=pltpu.SMEM),
        ],
        out_specs=pl.BlockSpec((BLK, DIM), lambda i: (i, 0)),
        out_shape=jax.ShapeDtypeStruct((N_NODES, DIM), jnp.float32),
    )(x, agg0, agg1, W1, b1.reshape(1, DIM), W2, b2.reshape(1, DIM),
      eps.reshape(1, 1))
    return out
